# Initial kernel scaffold; baseline (speedup 1.0000x reference)
import jax
import jax.numpy as jnp
from jax.experimental import pallas as pl


def kernel(hist_ids, wish_ids, bid, auth, lang, tags, dense, book_emb, auth_emb, lang_emb, tag_emb, dW1, db1, dW2, db2, uW1, ub1, uW2, ub2, uW3, ub3, uW4, ub4):
    raise NotImplementedError("write your pallas kernel here")


# speedup vs baseline: 6.3059x; 6.3059x over previous
import functools

import jax
import jax.numpy as jnp
from jax import lax
from jax.experimental import pallas as pl
from jax.experimental.pallas import tpu as pltpu
from jax.experimental.pallas import tpu_sc as plsc

B = 4096
D = 128
NH = 50
NWI = 20
NT = 10

NC = 2
NS = 16
NWRK = NC * NS
IPW = B // NWRK
LANE = 16

HCI = 8
WCI = 16
TCI = 32
HROWS = HCI * NH
WROWS = WCI * NWI
TROWS = TCI * NT
BUF_ROWS = 400

_mesh = plsc.VectorSubcoreMesh(core_axis_name="c", subcore_axis_name="s")


def _wid():
    return lax.axis_index("s") * NC + lax.axis_index("c")


def _acc_rows(buf, row_base, n_rows, out_ref, out_row, scale, init=None):
    nlane = D // LANE

    def body(r, acc):
        return tuple(
            acc[l] + buf[row_base + r, pl.ds(l * LANE, LANE)]
            for l in range(nlane)
        )

    acc0 = tuple(jnp.zeros((LANE,), jnp.float32) for _ in range(nlane))
    acc = lax.fori_loop(0, n_rows, body, acc0)
    for l in range(nlane):
        sl = pl.ds(l * LANE, LANE)
        v = acc[l] * scale
        if init == "add":
            out_ref[out_row, sl] = out_ref[out_row, sl] + v
        else:
            out_ref[out_row, sl] = v


@functools.partial(
    pl.kernel,
    out_type=jax.ShapeDtypeStruct((B, D), jnp.float32),
    mesh=_mesh,
    scratch_types=[
        pltpu.VMEM((IPW * NH,), jnp.int32),
        pltpu.VMEM((IPW * NWI,), jnp.int32),
        pltpu.VMEM((BUF_ROWS, D), jnp.float32),
        pltpu.VMEM((IPW, D), jnp.float32),
        pltpu.SemaphoreType.DMA,
    ],
)
def _user_pool(hist_hbm, wish_hbm, book_hbm, x_hbm, hidx, widx, buf, xacc, sem):
    wid = _wid()
    pltpu.sync_copy(hist_hbm.at[wid], hidx)
    pltpu.sync_copy(wish_hbm.at[wid], widx)

    @pl.loop(0, IPW // HCI)
    def _(c):
        pltpu.sync_copy(
            book_hbm.at[hidx.at[pl.ds(c * HROWS, HROWS)]],
            buf.at[pl.ds(0, HROWS)],
        )

        @pl.loop(0, HCI)
        def _(j):
            _acc_rows(buf, j * NH, NH, xacc, c * HCI + j, 1.0 / NH)

    @pl.loop(0, IPW // WCI)
    def _(c):
        pltpu.sync_copy(
            book_hbm.at[widx.at[pl.ds(c * WROWS, WROWS)]],
            buf.at[pl.ds(0, WROWS)],
        )

        @pl.loop(0, WCI)
        def _(j):
            _acc_rows(buf, j * NWI, NWI, xacc, c * WCI + j, 1.0 / NWI,
                      init="add")

    pltpu.sync_copy(xacc, x_hbm.at[pl.ds(wid * IPW, IPW)])


_T_OFF = 0
_B_OFF = IPW * NT
_A_OFF = _B_OFF + IPW
_L_OFF = _A_OFF + IPW
_ITM_W = _L_OFF + IPW


@functools.partial(
    pl.kernel,
    out_type=jax.ShapeDtypeStruct((B, D), jnp.float32),
    mesh=_mesh,
    scratch_types=[
        pltpu.VMEM((_ITM_W,), jnp.int32),
        pltpu.VMEM((BUF_ROWS, D), jnp.float32),
        pltpu.VMEM((IPW, D), jnp.float32),
        pltpu.SemaphoreType.DMA,
    ],
)
def _item_pool(itm_hbm, book_hbm, auth_hbm, lang_hbm, tag_hbm, i_hbm,
               iidx, buf, iacc, sem):
    wid = _wid()
    pltpu.sync_copy(itm_hbm.at[wid], iidx)

    pltpu.sync_copy(book_hbm.at[iidx.at[pl.ds(_B_OFF, IPW)]], iacc)

    for off, tbl in ((_A_OFF, auth_hbm), (_L_OFF, lang_hbm)):
        pltpu.sync_copy(tbl.at[iidx.at[pl.ds(off, IPW)]],
                        buf.at[pl.ds(0, IPW)])

        @pl.loop(0, IPW)
        def _(r):
            for l in range(D // LANE):
                sl = pl.ds(l * LANE, LANE)
                iacc[r, sl] = iacc[r, sl] + buf[r, sl]

    @pl.loop(0, IPW // TCI)
    def _(c):
        pltpu.sync_copy(
            tag_hbm.at[iidx.at[pl.ds(_T_OFF + c * TROWS, TROWS)]],
            buf.at[pl.ds(0, TROWS)],
        )

        @pl.loop(0, TCI)
        def _(j):
            _acc_rows(buf, j * NT, NT, iacc, c * TCI + j, 1.0 / NT,
                      init="add")

    pltpu.sync_copy(iacc, i_hbm.at[pl.ds(wid * IPW, IPW)])


_BM = 1024


def _mlp_body(x_ref, w1, b1, w2, b2, w3, b3, w4, b4, o_ref):
    f32 = jnp.float32
    h = jnp.maximum(jnp.dot(x_ref[...], w1[...], preferred_element_type=f32)
                    + b1[...], 0.0)
    h = jnp.maximum(jnp.dot(h, w2[...], preferred_element_type=f32)
                    + b2[...], 0.0)
    h = jnp.maximum(jnp.dot(h, w3[...], preferred_element_type=f32)
                    + b3[...], 0.0)
    o_ref[...] = jnp.dot(h, w4[...], preferred_element_type=f32) + b4[...]


def _user_mlp(x, uW1, ub1, uW2, ub2, uW3, ub3, uW4, ub4):
    full = lambda s: pl.BlockSpec(s, lambda i: (0, 0))
    return pl.pallas_call(
        _mlp_body,
        grid=(B // _BM,),
        in_specs=[
            pl.BlockSpec((_BM, D), lambda i: (i, 0)),
            full(uW1.shape), full(ub1.shape),
            full(uW2.shape), full(ub2.shape),
            full(uW3.shape), full(ub3.shape),
            full(uW4.shape), full(ub4.shape),
        ],
        out_specs=pl.BlockSpec((_BM, D), lambda i: (i, 0)),
        out_shape=jax.ShapeDtypeStruct((B, D), jnp.float32),
        compiler_params=pltpu.CompilerParams(
            dimension_semantics=("arbitrary",)),
    )(x, uW1, ub1, uW2, ub2, uW3, ub3, uW4, ub4)


def _combine_body(u_ref, ip_ref, dn_ref, w1, b1, w2, b2, o_ref):
    f32 = jnp.float32
    h = jnp.maximum(jnp.dot(dn_ref[...], w1[...], preferred_element_type=f32)
                    + b1[...], 0.0)
    d = jnp.dot(h, w2[...], preferred_element_type=f32) + b2[...]
    o_ref[...] = jnp.sum(u_ref[...] * (ip_ref[...] + d), axis=1,
                         keepdims=True)


def _combine(u, ipart, dense8, dW1p, db1, dW2, db2):
    full = lambda s: pl.BlockSpec(s, lambda i: (0, 0))
    return pl.pallas_call(
        _combine_body,
        grid=(B // _BM,),
        in_specs=[
            pl.BlockSpec((_BM, D), lambda i: (i, 0)),
            pl.BlockSpec((_BM, D), lambda i: (i, 0)),
            pl.BlockSpec((_BM, 8), lambda i: (i, 0)),
            full(dW1p.shape), full(db1.shape),
            full(dW2.shape), full(db2.shape),
        ],
        out_specs=pl.BlockSpec((_BM, 1), lambda i: (i, 0)),
        out_shape=jax.ShapeDtypeStruct((B, 1), jnp.float32),
        compiler_params=pltpu.CompilerParams(
            dimension_semantics=("arbitrary",)),
    )(u, ipart, dense8, dW1p, db1, dW2, db2)


def kernel(hist_ids, wish_ids, bid, auth, lang, tags, dense, book_emb,
           auth_emb, lang_emb, tag_emb, dW1, db1, dW2, db2, uW1, ub1,
           uW2, ub2, uW3, ub3, uW4, ub4):
    i32 = jnp.int32
    hist_r = hist_ids.astype(i32).reshape(NWRK, IPW * NH)
    wish_r = wish_ids.astype(i32).reshape(NWRK, IPW * NWI)
    itm = jnp.concatenate(
        [tags.astype(i32).reshape(NWRK, IPW * NT),
         bid.astype(i32).reshape(NWRK, IPW),
         auth.astype(i32).reshape(NWRK, IPW),
         lang.astype(i32).reshape(NWRK, IPW)],
        axis=1,
    )

    x = _user_pool(hist_r, wish_r, book_emb)
    ipart = _item_pool(itm, book_emb, auth_emb, lang_emb, tag_emb)

    u = _user_mlp(x, uW1, ub1.reshape(1, -1), uW2, ub2.reshape(1, -1),
                  uW3, ub3.reshape(1, -1), uW4, ub4.reshape(1, -1))

    dense8 = jnp.pad(dense, ((0, 0), (0, 5)))
    dW1p = jnp.pad(dW1, ((0, 5), (0, 0)))
    return _combine(u, ipart, dense8, dW1p, db1.reshape(1, -1), dW2,
                    db2.reshape(1, -1))


# speedup vs baseline: 8.0278x; 1.2731x over previous
import functools

import jax
import jax.numpy as jnp
from jax import lax
from jax.experimental import pallas as pl
from jax.experimental.pallas import tpu as pltpu
from jax.experimental.pallas import tpu_sc as plsc

B = 4096
D = 128
NH = 50
NWI = 20
NT = 10

NC = 2
NS = 16
NWRK = NC * NS
IPW = B // NWRK
LANE = 16

HCI = 4
WCI = 8
TCI = 16
HROWS = HCI * NH
WROWS = WCI * NWI
TROWS = TCI * NT
BUF_ROWS = 200

_mesh = plsc.VectorSubcoreMesh(core_axis_name="c", subcore_axis_name="s")


def _wid():
    return lax.axis_index("s") * NC + lax.axis_index("c")


def _acc_rows(buf, row_base, n_rows, out_ref, out_row, scale, init=None):
    nlane = D // LANE

    def body(r, acc):
        return tuple(
            acc[l] + buf[row_base + r, pl.ds(l * LANE, LANE)]
            for l in range(nlane)
        )

    acc0 = tuple(jnp.zeros((LANE,), jnp.float32) for _ in range(nlane))
    acc = lax.fori_loop(0, n_rows, body, acc0)
    for l in range(nlane):
        sl = pl.ds(l * LANE, LANE)
        v = acc[l] * scale
        if init == "add":
            out_ref[out_row, sl] = out_ref[out_row, sl] + v
        else:
            out_ref[out_row, sl] = v


@functools.partial(
    pl.kernel,
    out_type=jax.ShapeDtypeStruct((B, D), jnp.float32),
    mesh=_mesh,
    scratch_types=[
        pltpu.VMEM((IPW * NH,), jnp.int32),
        pltpu.VMEM((IPW * NWI,), jnp.int32),
        pltpu.VMEM((BUF_ROWS, D), jnp.float32),
        pltpu.VMEM((BUF_ROWS, D), jnp.float32),
        pltpu.VMEM((IPW, D), jnp.float32),
        pltpu.SemaphoreType.DMA,
        pltpu.SemaphoreType.DMA,
    ],
)
def _user_pool(hist_hbm, wish_hbm, book_hbm, x_hbm, hidx, widx, buf0, buf1,
               xacc, sem0, sem1):
    wid = _wid()
    pltpu.sync_copy(hist_hbm.at[wid], hidx)
    pltpu.sync_copy(wish_hbm.at[wid], widx)
    bufs = (buf0, buf1)
    sems = (sem0, sem1)

    def start(idx_slab, c, rows, b):
        pltpu.async_copy(
            book_hbm.at[idx_slab.at[pl.ds(c * rows, rows)]],
            bufs[b].at[pl.ds(0, rows)], sems[b])

    def wait(idx_slab, rows, b):
        pltpu.make_async_copy(
            book_hbm.at[idx_slab.at[pl.ds(0, rows)]],
            bufs[b].at[pl.ds(0, rows)], sems[b]).wait()

    NCH = IPW // HCI
    NCW = IPW // WCI

    start(hidx, 0, HROWS, 0)

    @pl.loop(0, NCH, step=2)
    def _(c):
        for b in range(2):
            cc = c + b

            @pl.when(cc + 1 < NCH)
            def _():
                start(hidx, cc + 1, HROWS, 1 - b)

            wait(hidx, HROWS, b)

            @pl.loop(0, HCI)
            def _(j):
                _acc_rows(bufs[b], j * NH, NH, xacc, cc * HCI + j, 1.0 / NH)

    start(widx, 0, WROWS, 0)

    @pl.loop(0, NCW, step=2)
    def _(c):
        for b in range(2):
            cc = c + b

            @pl.when(cc + 1 < NCW)
            def _():
                start(widx, cc + 1, WROWS, 1 - b)

            wait(widx, WROWS, b)

            @pl.loop(0, WCI)
            def _(j):
                _acc_rows(bufs[b], j * NWI, NWI, xacc, cc * WCI + j,
                          1.0 / NWI, init="add")

    pltpu.sync_copy(xacc, x_hbm.at[pl.ds(wid * IPW, IPW)])


_T_OFF = 0
_B_OFF = IPW * NT
_A_OFF = _B_OFF + IPW
_L_OFF = _A_OFF + IPW
_ITM_W = _L_OFF + IPW


@functools.partial(
    pl.kernel,
    out_type=jax.ShapeDtypeStruct((B, D), jnp.float32),
    mesh=_mesh,
    scratch_types=[
        pltpu.VMEM((_ITM_W,), jnp.int32),
        pltpu.VMEM((BUF_ROWS, D), jnp.float32),
        pltpu.VMEM((IPW, D), jnp.float32),
        pltpu.SemaphoreType.DMA,
    ],
)
def _item_pool(itm_hbm, book_hbm, auth_hbm, lang_hbm, tag_hbm, i_hbm,
               iidx, buf, iacc, sem):
    wid = _wid()
    pltpu.sync_copy(itm_hbm.at[wid], iidx)

    pltpu.sync_copy(book_hbm.at[iidx.at[pl.ds(_B_OFF, IPW)]], iacc)

    for off, tbl in ((_A_OFF, auth_hbm), (_L_OFF, lang_hbm)):
        pltpu.sync_copy(tbl.at[iidx.at[pl.ds(off, IPW)]],
                        buf.at[pl.ds(0, IPW)])

        @pl.loop(0, IPW)
        def _(r):
            for l in range(D // LANE):
                sl = pl.ds(l * LANE, LANE)
                iacc[r, sl] = iacc[r, sl] + buf[r, sl]

    @pl.loop(0, IPW // TCI)
    def _(c):
        pltpu.sync_copy(
            tag_hbm.at[iidx.at[pl.ds(_T_OFF + c * TROWS, TROWS)]],
            buf.at[pl.ds(0, TROWS)],
        )

        @pl.loop(0, TCI)
        def _(j):
            _acc_rows(buf, j * NT, NT, iacc, c * TCI + j, 1.0 / NT,
                      init="add")

    pltpu.sync_copy(iacc, i_hbm.at[pl.ds(wid * IPW, IPW)])


_BM = 1024


def _mlp_body(x_ref, w1, b1, w2, b2, w3, b3, w4, b4, o_ref):
    f32 = jnp.float32
    h = jnp.maximum(jnp.dot(x_ref[...], w1[...], preferred_element_type=f32)
                    + b1[...], 0.0)
    h = jnp.maximum(jnp.dot(h, w2[...], preferred_element_type=f32)
                    + b2[...], 0.0)
    h = jnp.maximum(jnp.dot(h, w3[...], preferred_element_type=f32)
                    + b3[...], 0.0)
    o_ref[...] = jnp.dot(h, w4[...], preferred_element_type=f32) + b4[...]


def _user_mlp(x, uW1, ub1, uW2, ub2, uW3, ub3, uW4, ub4):
    full = lambda s: pl.BlockSpec(s, lambda i: (0, 0))
    return pl.pallas_call(
        _mlp_body,
        grid=(B // _BM,),
        in_specs=[
            pl.BlockSpec((_BM, D), lambda i: (i, 0)),
            full(uW1.shape), full(ub1.shape),
            full(uW2.shape), full(ub2.shape),
            full(uW3.shape), full(ub3.shape),
            full(uW4.shape), full(ub4.shape),
        ],
        out_specs=pl.BlockSpec((_BM, D), lambda i: (i, 0)),
        out_shape=jax.ShapeDtypeStruct((B, D), jnp.float32),
        compiler_params=pltpu.CompilerParams(
            dimension_semantics=("arbitrary",)),
    )(x, uW1, ub1, uW2, ub2, uW3, ub3, uW4, ub4)


def _combine_body(u_ref, ip_ref, dn_ref, w1, b1, w2, b2, o_ref):
    f32 = jnp.float32
    h = jnp.maximum(jnp.dot(dn_ref[...], w1[...], preferred_element_type=f32)
                    + b1[...], 0.0)
    d = jnp.dot(h, w2[...], preferred_element_type=f32) + b2[...]
    o_ref[...] = jnp.sum(u_ref[...] * (ip_ref[...] + d), axis=1,
                         keepdims=True)


def _combine(u, ipart, dense8, dW1p, db1, dW2, db2):
    full = lambda s: pl.BlockSpec(s, lambda i: (0, 0))
    return pl.pallas_call(
        _combine_body,
        grid=(B // _BM,),
        in_specs=[
            pl.BlockSpec((_BM, D), lambda i: (i, 0)),
            pl.BlockSpec((_BM, D), lambda i: (i, 0)),
            pl.BlockSpec((_BM, 8), lambda i: (i, 0)),
            full(dW1p.shape), full(db1.shape),
            full(dW2.shape), full(db2.shape),
        ],
        out_specs=pl.BlockSpec((_BM, 1), lambda i: (i, 0)),
        out_shape=jax.ShapeDtypeStruct((B, 1), jnp.float32),
        compiler_params=pltpu.CompilerParams(
            dimension_semantics=("arbitrary",)),
    )(u, ipart, dense8, dW1p, db1, dW2, db2)


def kernel(hist_ids, wish_ids, bid, auth, lang, tags, dense, book_emb,
           auth_emb, lang_emb, tag_emb, dW1, db1, dW2, db2, uW1, ub1,
           uW2, ub2, uW3, ub3, uW4, ub4):
    i32 = jnp.int32
    hist_r = hist_ids.astype(i32).reshape(NWRK, IPW * NH)
    wish_r = wish_ids.astype(i32).reshape(NWRK, IPW * NWI)
    itm = jnp.concatenate(
        [tags.astype(i32).reshape(NWRK, IPW * NT),
         bid.astype(i32).reshape(NWRK, IPW),
         auth.astype(i32).reshape(NWRK, IPW),
         lang.astype(i32).reshape(NWRK, IPW)],
        axis=1,
    )

    x = _user_pool(hist_r, wish_r, book_emb)
    ipart = _item_pool(itm, book_emb, auth_emb, lang_emb, tag_emb)

    u = _user_mlp(x, uW1, ub1.reshape(1, -1), uW2, ub2.reshape(1, -1),
                  uW3, ub3.reshape(1, -1), uW4, ub4.reshape(1, -1))

    dense8 = jnp.pad(dense, ((0, 0), (0, 5)))
    dW1p = jnp.pad(dW1, ((0, 5), (0, 0)))
    return _combine(u, ipart, dense8, dW1p, db1.reshape(1, -1), dW2,
                    db2.reshape(1, -1))


# speedup vs baseline: 8.8082x; 1.0972x over previous
import functools

import jax
import jax.numpy as jnp
from jax import lax
from jax.experimental import pallas as pl
from jax.experimental.pallas import tpu as pltpu
from jax.experimental.pallas import tpu_sc as plsc

B = 4096
D = 128
NH = 50
NWI = 20
NT = 10

NC = 2
NS = 16
NWRK = NC * NS
IPW = B // NWRK
LANE = 16

HCI = 4
WCI = 8
TCI = 16
HROWS = HCI * NH
WROWS = WCI * NWI
TROWS = TCI * NT
BUF_ROWS = 200

_mesh = plsc.VectorSubcoreMesh(core_axis_name="c", subcore_axis_name="s")


def _wid():
    return lax.axis_index("s") * NC + lax.axis_index("c")


def _acc_rows(buf, row_base, n_rows, out_ref, out_row, scale, init=None):
    nlane = D // LANE

    def body(r, acc):
        return tuple(
            acc[l] + buf[row_base + r, pl.ds(l * LANE, LANE)]
            for l in range(nlane)
        )

    acc0 = tuple(jnp.zeros((LANE,), jnp.float32) for _ in range(nlane))
    acc = lax.fori_loop(0, n_rows, body, acc0)
    for l in range(nlane):
        sl = pl.ds(l * LANE, LANE)
        v = acc[l] * scale
        if init == "add":
            out_ref[out_row, sl] = out_ref[out_row, sl] + v
        else:
            out_ref[out_row, sl] = v


@functools.partial(
    pl.kernel,
    out_type=jax.ShapeDtypeStruct((B, D), jnp.float32),
    mesh=_mesh,
    scratch_types=[
        pltpu.VMEM((IPW * NH,), jnp.int32),
        pltpu.VMEM((IPW * NWI,), jnp.int32),
        pltpu.VMEM((BUF_ROWS, D), jnp.float32),
        pltpu.VMEM((BUF_ROWS, D), jnp.float32),
        pltpu.VMEM((IPW, D), jnp.float32),
        pltpu.SemaphoreType.DMA,
        pltpu.SemaphoreType.DMA,
    ],
)
def _user_pool(hist_hbm, wish_hbm, book_hbm, x_hbm, hidx, widx, buf0, buf1,
               xacc, sem0, sem1):
    wid = _wid()
    pltpu.sync_copy(hist_hbm.at[wid], hidx)
    pltpu.sync_copy(wish_hbm.at[wid], widx)
    bufs = (buf0, buf1)
    sems = (sem0, sem1)

    def start(idx_slab, c, rows, b):
        pltpu.async_copy(
            book_hbm.at[idx_slab.at[pl.ds(c * rows, rows)]],
            bufs[b].at[pl.ds(0, rows)], sems[b])

    def wait(idx_slab, rows, b):
        pltpu.make_async_copy(
            book_hbm.at[idx_slab.at[pl.ds(0, rows)]],
            bufs[b].at[pl.ds(0, rows)], sems[b]).wait()

    NCH = IPW // HCI
    NCW = IPW // WCI

    start(hidx, 0, HROWS, 0)

    @pl.loop(0, NCH, step=2)
    def _(c):
        for b in range(2):
            cc = c + b

            @pl.when(cc + 1 < NCH)
            def _():
                start(hidx, cc + 1, HROWS, 1 - b)

            wait(hidx, HROWS, b)

            @pl.loop(0, HCI)
            def _(j):
                _acc_rows(bufs[b], j * NH, NH, xacc, cc * HCI + j, 1.0 / NH)

    start(widx, 0, WROWS, 0)

    @pl.loop(0, NCW, step=2)
    def _(c):
        for b in range(2):
            cc = c + b

            @pl.when(cc + 1 < NCW)
            def _():
                start(widx, cc + 1, WROWS, 1 - b)

            wait(widx, WROWS, b)

            @pl.loop(0, WCI)
            def _(j):
                _acc_rows(bufs[b], j * NWI, NWI, xacc, cc * WCI + j,
                          1.0 / NWI, init="add")

    pltpu.sync_copy(xacc, x_hbm.at[pl.ds(wid * IPW, IPW)])


_T_OFF = 0
_B_OFF = IPW * NT
_A_OFF = _B_OFF + IPW
_L_OFF = _A_OFF + IPW
_ITM_W = _L_OFF + IPW


@functools.partial(
    pl.kernel,
    out_type=jax.ShapeDtypeStruct((B, D), jnp.float32),
    mesh=_mesh,
    scratch_types=[
        pltpu.VMEM((_ITM_W,), jnp.int32),
        pltpu.VMEM((BUF_ROWS, D), jnp.float32),
        pltpu.VMEM((BUF_ROWS, D), jnp.float32),
        pltpu.VMEM((IPW, D), jnp.float32),
        pltpu.SemaphoreType.DMA,
        pltpu.SemaphoreType.DMA,
        pltpu.SemaphoreType.DMA,
    ],
)
def _item_pool(itm_hbm, book_hbm, auth_hbm, lang_hbm, tag_hbm, i_hbm,
               iidx, buf0, buf1, iacc, sem0, sem1, sem2):
    wid = _wid()
    pltpu.sync_copy(itm_hbm.at[wid], iidx)
    bufs = (buf0, buf1)
    sems = (sem0, sem1)

    pltpu.async_copy(book_hbm.at[iidx.at[pl.ds(_B_OFF, IPW)]], iacc, sem2)
    pltpu.async_copy(auth_hbm.at[iidx.at[pl.ds(_A_OFF, IPW)]],
                     buf0.at[pl.ds(0, IPW)], sem0)
    pltpu.async_copy(lang_hbm.at[iidx.at[pl.ds(_L_OFF, IPW)]],
                     buf1.at[pl.ds(0, IPW)], sem1)

    def add_buf(b):
        @pl.loop(0, IPW)
        def _(r):
            for l in range(D // LANE):
                sl = pl.ds(l * LANE, LANE)
                iacc[r, sl] = iacc[r, sl] + bufs[b][r, sl]

    def start_t(c, b):
        pltpu.async_copy(
            tag_hbm.at[iidx.at[pl.ds(_T_OFF + c * TROWS, TROWS)]],
            bufs[b].at[pl.ds(0, TROWS)], sems[b])

    def wait_t(b):
        pltpu.make_async_copy(
            tag_hbm.at[iidx.at[pl.ds(0, TROWS)]],
            bufs[b].at[pl.ds(0, TROWS)], sems[b]).wait()

    NCT = IPW // TCI

    pltpu.make_async_copy(book_hbm.at[iidx.at[pl.ds(_B_OFF, IPW)]], iacc,
                          sem2).wait()
    pltpu.make_async_copy(auth_hbm.at[iidx.at[pl.ds(_A_OFF, IPW)]],
                          buf0.at[pl.ds(0, IPW)], sem0).wait()
    add_buf(0)
    start_t(0, 0)
    pltpu.make_async_copy(lang_hbm.at[iidx.at[pl.ds(_L_OFF, IPW)]],
                          buf1.at[pl.ds(0, IPW)], sem1).wait()
    add_buf(1)
    start_t(1, 1)

    @pl.loop(0, NCT, step=2)
    def _(c):
        for b in range(2):
            cc = c + b
            wait_t(b)

            @pl.loop(0, TCI)
            def _(j):
                _acc_rows(bufs[b], j * NT, NT, iacc, cc * TCI + j, 1.0 / NT,
                          init="add")

            @pl.when(cc + 2 < NCT)
            def _():
                start_t(cc + 2, b)

    pltpu.sync_copy(iacc, i_hbm.at[pl.ds(wid * IPW, IPW)])


_BM = 1024


def _mlp_body(x_ref, w1, b1, w2, b2, w3, b3, w4, b4, o_ref):
    f32 = jnp.float32
    h = jnp.maximum(jnp.dot(x_ref[...], w1[...], preferred_element_type=f32)
                    + b1[...], 0.0)
    h = jnp.maximum(jnp.dot(h, w2[...], preferred_element_type=f32)
                    + b2[...], 0.0)
    h = jnp.maximum(jnp.dot(h, w3[...], preferred_element_type=f32)
                    + b3[...], 0.0)
    o_ref[...] = jnp.dot(h, w4[...], preferred_element_type=f32) + b4[...]


def _user_mlp(x, uW1, ub1, uW2, ub2, uW3, ub3, uW4, ub4):
    full = lambda s: pl.BlockSpec(s, lambda i: (0, 0))
    return pl.pallas_call(
        _mlp_body,
        grid=(B // _BM,),
        in_specs=[
            pl.BlockSpec((_BM, D), lambda i: (i, 0)),
            full(uW1.shape), full(ub1.shape),
            full(uW2.shape), full(ub2.shape),
            full(uW3.shape), full(ub3.shape),
            full(uW4.shape), full(ub4.shape),
        ],
        out_specs=pl.BlockSpec((_BM, D), lambda i: (i, 0)),
        out_shape=jax.ShapeDtypeStruct((B, D), jnp.float32),
        compiler_params=pltpu.CompilerParams(
            dimension_semantics=("arbitrary",)),
    )(x, uW1, ub1, uW2, ub2, uW3, ub3, uW4, ub4)


def _combine_body(u_ref, ip_ref, dn_ref, w1, b1, w2, b2, o_ref):
    f32 = jnp.float32
    h = jnp.maximum(jnp.dot(dn_ref[...], w1[...], preferred_element_type=f32)
                    + b1[...], 0.0)
    d = jnp.dot(h, w2[...], preferred_element_type=f32) + b2[...]
    o_ref[...] = jnp.sum(u_ref[...] * (ip_ref[...] + d), axis=1,
                         keepdims=True)


def _combine(u, ipart, dense8, dW1p, db1, dW2, db2):
    full = lambda s: pl.BlockSpec(s, lambda i: (0, 0))
    return pl.pallas_call(
        _combine_body,
        grid=(B // _BM,),
        in_specs=[
            pl.BlockSpec((_BM, D), lambda i: (i, 0)),
            pl.BlockSpec((_BM, D), lambda i: (i, 0)),
            pl.BlockSpec((_BM, 8), lambda i: (i, 0)),
            full(dW1p.shape), full(db1.shape),
            full(dW2.shape), full(db2.shape),
        ],
        out_specs=pl.BlockSpec((_BM, 1), lambda i: (i, 0)),
        out_shape=jax.ShapeDtypeStruct((B, 1), jnp.float32),
        compiler_params=pltpu.CompilerParams(
            dimension_semantics=("arbitrary",)),
    )(u, ipart, dense8, dW1p, db1, dW2, db2)


def kernel(hist_ids, wish_ids, bid, auth, lang, tags, dense, book_emb,
           auth_emb, lang_emb, tag_emb, dW1, db1, dW2, db2, uW1, ub1,
           uW2, ub2, uW3, ub3, uW4, ub4):
    i32 = jnp.int32
    hist_r = hist_ids.astype(i32).reshape(NWRK, IPW * NH)
    wish_r = wish_ids.astype(i32).reshape(NWRK, IPW * NWI)
    itm = jnp.concatenate(
        [tags.astype(i32).reshape(NWRK, IPW * NT),
         bid.astype(i32).reshape(NWRK, IPW),
         auth.astype(i32).reshape(NWRK, IPW),
         lang.astype(i32).reshape(NWRK, IPW)],
        axis=1,
    )

    x = _user_pool(hist_r, wish_r, book_emb)
    ipart = _item_pool(itm, book_emb, auth_emb, lang_emb, tag_emb)

    u = _user_mlp(x, uW1, ub1.reshape(1, -1), uW2, ub2.reshape(1, -1),
                  uW3, ub3.reshape(1, -1), uW4, ub4.reshape(1, -1))

    dense8 = jnp.pad(dense, ((0, 0), (0, 5)))
    dW1p = jnp.pad(dW1, ((0, 5), (0, 0)))
    return _combine(u, ipart, dense8, dW1p, db1.reshape(1, -1), dW2,
                    db2.reshape(1, -1))


# speedup vs baseline: 10.4958x; 1.1916x over previous
import functools

import jax
import jax.numpy as jnp
from jax import lax
from jax.experimental import pallas as pl
from jax.experimental.pallas import tpu as pltpu
from jax.experimental.pallas import tpu_sc as plsc

B = 4096
D = 128
NH = 50
NWI = 20
NT = 10

NC = 2
NS = 16
NWRK = NC * NS
IPW = B // NWRK
LANE = 16

HCI = 4
WCI = 8
TCI = 16
HROWS = HCI * NH
WROWS = WCI * NWI
TROWS = TCI * NT
BUF_ROWS = 200

_mesh = plsc.VectorSubcoreMesh(core_axis_name="c", subcore_axis_name="s")


def _wid():
    return lax.axis_index("s") * NC + lax.axis_index("c")


def _acc_rows(buf, row_base, n_rows, out_ref, out_row, scale, init=None):
    nlane = D // LANE

    def body(r, acc):
        return tuple(
            acc[l] + buf[row_base + r, pl.ds(l * LANE, LANE)]
            for l in range(nlane)
        )

    acc0 = tuple(jnp.zeros((LANE,), jnp.float32) for _ in range(nlane))
    acc = lax.fori_loop(0, n_rows, body, acc0)
    for l in range(nlane):
        sl = pl.ds(l * LANE, LANE)
        v = acc[l] * scale
        if init == "add":
            out_ref[out_row, sl] = out_ref[out_row, sl] + v
        else:
            out_ref[out_row, sl] = v


@functools.partial(
    pl.kernel,
    out_type=jax.ShapeDtypeStruct((B, D), jnp.float32),
    mesh=_mesh,
    scratch_types=[
        pltpu.VMEM((NH * IPW,), jnp.int32),
        pltpu.VMEM((NWI * IPW,), jnp.int32),
        pltpu.VMEM((IPW, D), jnp.float32),
        pltpu.VMEM((IPW, D), jnp.float32),
        pltpu.SemaphoreType.DMA,
        pltpu.SemaphoreType.DMA,
    ],
)
def _user_pool(hist_hbm, wish_hbm, book_hbm, x_hbm, hidx, widx, hacc, wacc,
               hsem, wsem):
    wid = _wid()
    pltpu.sync_copy(hist_hbm.at[wid], hidx)
    pltpu.sync_copy(wish_hbm.at[wid], widx)

    zeros = jnp.zeros((LANE,), jnp.float32)

    @pl.loop(0, IPW)
    def _(r):
        for l in range(D // LANE):
            sl = pl.ds(l * LANE, LANE)
            hacc[r, sl] = zeros
            wacc[r, sl] = zeros

    @pl.loop(0, NH)
    def _(k):
        pltpu.async_copy(book_hbm.at[hidx.at[pl.ds(k * IPW, IPW)]], hacc,
                         hsem, add=True)

    @pl.loop(0, NWI)
    def _(k):
        pltpu.async_copy(book_hbm.at[widx.at[pl.ds(k * IPW, IPW)]], wacc,
                         wsem, add=True)

    @pl.loop(0, NH)
    def _(k):
        pltpu.make_async_copy(book_hbm.at[hidx.at[pl.ds(0, IPW)]], hacc,
                              hsem).wait()

    @pl.loop(0, NWI)
    def _(k):
        pltpu.make_async_copy(book_hbm.at[widx.at[pl.ds(0, IPW)]], wacc,
                              wsem).wait()

    @pl.loop(0, IPW)
    def _(r):
        for l in range(D // LANE):
            sl = pl.ds(l * LANE, LANE)
            hacc[r, sl] = hacc[r, sl] * (1.0 / NH) + wacc[r, sl] * (1.0 / NWI)

    pltpu.sync_copy(hacc, x_hbm.at[pl.ds(wid * IPW, IPW)])


_T_OFF = 0
_B_OFF = IPW * NT
_A_OFF = _B_OFF + IPW
_L_OFF = _A_OFF + IPW
_ITM_W = _L_OFF + IPW


@functools.partial(
    pl.kernel,
    out_type=jax.ShapeDtypeStruct((B, D), jnp.float32),
    mesh=_mesh,
    scratch_types=[
        pltpu.VMEM((_ITM_W,), jnp.int32),
        pltpu.VMEM((BUF_ROWS, D), jnp.float32),
        pltpu.VMEM((BUF_ROWS, D), jnp.float32),
        pltpu.VMEM((IPW, D), jnp.float32),
        pltpu.SemaphoreType.DMA,
        pltpu.SemaphoreType.DMA,
        pltpu.SemaphoreType.DMA,
    ],
)
def _item_pool(itm_hbm, book_hbm, auth_hbm, lang_hbm, tag_hbm, i_hbm,
               iidx, buf0, buf1, iacc, sem0, sem1, sem2):
    wid = _wid()
    pltpu.sync_copy(itm_hbm.at[wid], iidx)
    bufs = (buf0, buf1)
    sems = (sem0, sem1)

    pltpu.async_copy(book_hbm.at[iidx.at[pl.ds(_B_OFF, IPW)]], iacc, sem2)
    pltpu.async_copy(auth_hbm.at[iidx.at[pl.ds(_A_OFF, IPW)]],
                     buf0.at[pl.ds(0, IPW)], sem0)
    pltpu.async_copy(lang_hbm.at[iidx.at[pl.ds(_L_OFF, IPW)]],
                     buf1.at[pl.ds(0, IPW)], sem1)

    def add_buf(b):
        @pl.loop(0, IPW)
        def _(r):
            for l in range(D // LANE):
                sl = pl.ds(l * LANE, LANE)
                iacc[r, sl] = iacc[r, sl] + bufs[b][r, sl]

    def start_t(c, b):
        pltpu.async_copy(
            tag_hbm.at[iidx.at[pl.ds(_T_OFF + c * TROWS, TROWS)]],
            bufs[b].at[pl.ds(0, TROWS)], sems[b])

    def wait_t(b):
        pltpu.make_async_copy(
            tag_hbm.at[iidx.at[pl.ds(0, TROWS)]],
            bufs[b].at[pl.ds(0, TROWS)], sems[b]).wait()

    NCT = IPW // TCI

    pltpu.make_async_copy(book_hbm.at[iidx.at[pl.ds(_B_OFF, IPW)]], iacc,
                          sem2).wait()
    pltpu.make_async_copy(auth_hbm.at[iidx.at[pl.ds(_A_OFF, IPW)]],
                          buf0.at[pl.ds(0, IPW)], sem0).wait()
    add_buf(0)
    start_t(0, 0)
    pltpu.make_async_copy(lang_hbm.at[iidx.at[pl.ds(_L_OFF, IPW)]],
                          buf1.at[pl.ds(0, IPW)], sem1).wait()
    add_buf(1)
    start_t(1, 1)

    @pl.loop(0, NCT, step=2)
    def _(c):
        for b in range(2):
            cc = c + b
            wait_t(b)

            @pl.loop(0, TCI)
            def _(j):
                _acc_rows(bufs[b], j * NT, NT, iacc, cc * TCI + j, 1.0 / NT,
                          init="add")

            @pl.when(cc + 2 < NCT)
            def _():
                start_t(cc + 2, b)

    pltpu.sync_copy(iacc, i_hbm.at[pl.ds(wid * IPW, IPW)])


_BM = 1024


def _mlp_body(x_ref, w1, b1, w2, b2, w3, b3, w4, b4, o_ref):
    f32 = jnp.float32
    h = jnp.maximum(jnp.dot(x_ref[...], w1[...], preferred_element_type=f32)
                    + b1[...], 0.0)
    h = jnp.maximum(jnp.dot(h, w2[...], preferred_element_type=f32)
                    + b2[...], 0.0)
    h = jnp.maximum(jnp.dot(h, w3[...], preferred_element_type=f32)
                    + b3[...], 0.0)
    o_ref[...] = jnp.dot(h, w4[...], preferred_element_type=f32) + b4[...]


def _user_mlp(x, uW1, ub1, uW2, ub2, uW3, ub3, uW4, ub4):
    full = lambda s: pl.BlockSpec(s, lambda i: (0, 0))
    return pl.pallas_call(
        _mlp_body,
        grid=(B // _BM,),
        in_specs=[
            pl.BlockSpec((_BM, D), lambda i: (i, 0)),
            full(uW1.shape), full(ub1.shape),
            full(uW2.shape), full(ub2.shape),
            full(uW3.shape), full(ub3.shape),
            full(uW4.shape), full(ub4.shape),
        ],
        out_specs=pl.BlockSpec((_BM, D), lambda i: (i, 0)),
        out_shape=jax.ShapeDtypeStruct((B, D), jnp.float32),
        compiler_params=pltpu.CompilerParams(
            dimension_semantics=("arbitrary",)),
    )(x, uW1, ub1, uW2, ub2, uW3, ub3, uW4, ub4)


def _combine_body(u_ref, ip_ref, dn_ref, w1, b1, w2, b2, o_ref):
    f32 = jnp.float32
    h = jnp.maximum(jnp.dot(dn_ref[...], w1[...], preferred_element_type=f32)
                    + b1[...], 0.0)
    d = jnp.dot(h, w2[...], preferred_element_type=f32) + b2[...]
    o_ref[...] = jnp.sum(u_ref[...] * (ip_ref[...] + d), axis=1,
                         keepdims=True)


def _combine(u, ipart, dense8, dW1p, db1, dW2, db2):
    full = lambda s: pl.BlockSpec(s, lambda i: (0, 0))
    return pl.pallas_call(
        _combine_body,
        grid=(B // _BM,),
        in_specs=[
            pl.BlockSpec((_BM, D), lambda i: (i, 0)),
            pl.BlockSpec((_BM, D), lambda i: (i, 0)),
            pl.BlockSpec((_BM, 8), lambda i: (i, 0)),
            full(dW1p.shape), full(db1.shape),
            full(dW2.shape), full(db2.shape),
        ],
        out_specs=pl.BlockSpec((_BM, 1), lambda i: (i, 0)),
        out_shape=jax.ShapeDtypeStruct((B, 1), jnp.float32),
        compiler_params=pltpu.CompilerParams(
            dimension_semantics=("arbitrary",)),
    )(u, ipart, dense8, dW1p, db1, dW2, db2)


def kernel(hist_ids, wish_ids, bid, auth, lang, tags, dense, book_emb,
           auth_emb, lang_emb, tag_emb, dW1, db1, dW2, db2, uW1, ub1,
           uW2, ub2, uW3, ub3, uW4, ub4):
    i32 = jnp.int32
    hist_r = hist_ids.astype(i32).reshape(NWRK, IPW, NH).transpose(0, 2, 1) \
        .reshape(NWRK, NH * IPW)
    wish_r = wish_ids.astype(i32).reshape(NWRK, IPW, NWI).transpose(0, 2, 1) \
        .reshape(NWRK, NWI * IPW)
    itm = jnp.concatenate(
        [tags.astype(i32).reshape(NWRK, IPW * NT),
         bid.astype(i32).reshape(NWRK, IPW),
         auth.astype(i32).reshape(NWRK, IPW),
         lang.astype(i32).reshape(NWRK, IPW)],
        axis=1,
    )

    x = _user_pool(hist_r, wish_r, book_emb)
    ipart = _item_pool(itm, book_emb, auth_emb, lang_emb, tag_emb)

    u = _user_mlp(x, uW1, ub1.reshape(1, -1), uW2, ub2.reshape(1, -1),
                  uW3, ub3.reshape(1, -1), uW4, ub4.reshape(1, -1))

    dense8 = jnp.pad(dense, ((0, 0), (0, 5)))
    dW1p = jnp.pad(dW1, ((0, 5), (0, 0)))
    return _combine(u, ipart, dense8, dW1p, db1.reshape(1, -1), dW2,
                    db2.reshape(1, -1))


# speedup vs baseline: 10.9794x; 1.0461x over previous
import functools

import jax
import jax.numpy as jnp
from jax import lax
from jax.experimental import pallas as pl
from jax.experimental.pallas import tpu as pltpu
from jax.experimental.pallas import tpu_sc as plsc

B = 4096
D = 128
NH = 50
NWI = 20
NT = 10

NC = 2
NS = 16
NWRK = NC * NS
IPW = B // NWRK
LANE = 16

HCI = 4
WCI = 8
TCI = 16
HROWS = HCI * NH
WROWS = WCI * NWI
TROWS = TCI * NT
BUF_ROWS = 200

_mesh = plsc.VectorSubcoreMesh(core_axis_name="c", subcore_axis_name="s")


def _wid():
    return lax.axis_index("s") * NC + lax.axis_index("c")


def _acc_rows(buf, row_base, n_rows, out_ref, out_row, scale, init=None):
    nlane = D // LANE

    def body(r, acc):
        return tuple(
            acc[l] + buf[row_base + r, pl.ds(l * LANE, LANE)]
            for l in range(nlane)
        )

    acc0 = tuple(jnp.zeros((LANE,), jnp.float32) for _ in range(nlane))
    acc = lax.fori_loop(0, n_rows, body, acc0)
    for l in range(nlane):
        sl = pl.ds(l * LANE, LANE)
        v = acc[l] * scale
        if init == "add":
            out_ref[out_row, sl] = out_ref[out_row, sl] + v
        else:
            out_ref[out_row, sl] = v


@functools.partial(
    pl.kernel,
    out_type=jax.ShapeDtypeStruct((B, D), jnp.float32),
    mesh=_mesh,
    scratch_types=[
        pltpu.VMEM((NH * IPW,), jnp.int32),
        pltpu.VMEM((NWI * IPW,), jnp.int32),
        pltpu.VMEM((IPW, D), jnp.float32),
        pltpu.VMEM((IPW, D), jnp.float32),
        pltpu.SemaphoreType.DMA,
        pltpu.SemaphoreType.DMA,
    ],
)
def _user_pool(hist_hbm, wish_hbm, book_hbm, x_hbm, hidx, widx, hacc, wacc,
               hsem, wsem):
    wid = _wid()
    pltpu.sync_copy(hist_hbm.at[wid], hidx)
    pltpu.sync_copy(wish_hbm.at[wid], widx)

    zeros = jnp.zeros((LANE,), jnp.float32)

    @pl.loop(0, IPW)
    def _(r):
        for l in range(D // LANE):
            sl = pl.ds(l * LANE, LANE)
            hacc[r, sl] = zeros
            wacc[r, sl] = zeros

    @pl.loop(0, NH)
    def _(k):
        pltpu.async_copy(book_hbm.at[hidx.at[pl.ds(k * IPW, IPW)]], hacc,
                         hsem, add=True)

    @pl.loop(0, NWI)
    def _(k):
        pltpu.async_copy(book_hbm.at[widx.at[pl.ds(k * IPW, IPW)]], wacc,
                         wsem, add=True)

    @pl.loop(0, NH)
    def _(k):
        pltpu.make_async_copy(book_hbm.at[hidx.at[pl.ds(0, IPW)]], hacc,
                              hsem).wait()

    @pl.loop(0, NWI)
    def _(k):
        pltpu.make_async_copy(book_hbm.at[widx.at[pl.ds(0, IPW)]], wacc,
                              wsem).wait()

    @pl.loop(0, IPW)
    def _(r):
        for l in range(D // LANE):
            sl = pl.ds(l * LANE, LANE)
            hacc[r, sl] = hacc[r, sl] * (1.0 / NH) + wacc[r, sl] * (1.0 / NWI)

    pltpu.sync_copy(hacc, x_hbm.at[pl.ds(wid * IPW, IPW)])


_T_OFF = 0
_B_OFF = IPW * NT
_A_OFF = _B_OFF + IPW
_L_OFF = _A_OFF + IPW
_ITM_W = _L_OFF + IPW


@functools.partial(
    pl.kernel,
    out_type=jax.ShapeDtypeStruct((B, D), jnp.float32),
    mesh=_mesh,
    scratch_types=[
        pltpu.VMEM((_ITM_W,), jnp.int32),
        pltpu.VMEM((IPW, D), jnp.float32),
        pltpu.VMEM((IPW, D), jnp.float32),
        pltpu.SemaphoreType.DMA,
        pltpu.SemaphoreType.DMA,
    ],
)
def _item_pool(itm_hbm, book_hbm, auth_hbm, lang_hbm, tag_hbm, i_hbm,
               iidx, iacc, tacc, semi, semt):
    wid = _wid()
    pltpu.sync_copy(itm_hbm.at[wid], iidx)

    zeros = jnp.zeros((LANE,), jnp.float32)

    @pl.loop(0, IPW)
    def _(r):
        for l in range(D // LANE):
            sl = pl.ds(l * LANE, LANE)
            iacc[r, sl] = zeros
            tacc[r, sl] = zeros

    pltpu.async_copy(book_hbm.at[iidx.at[pl.ds(_B_OFF, IPW)]], iacc, semi,
                     add=True)
    pltpu.async_copy(auth_hbm.at[iidx.at[pl.ds(_A_OFF, IPW)]], iacc, semi,
                     add=True)
    pltpu.async_copy(lang_hbm.at[iidx.at[pl.ds(_L_OFF, IPW)]], iacc, semi,
                     add=True)

    @pl.loop(0, NT)
    def _(k):
        pltpu.async_copy(tag_hbm.at[iidx.at[pl.ds(_T_OFF + k * IPW, IPW)]],
                         tacc, semt, add=True)

    for _ in range(3):
        pltpu.make_async_copy(book_hbm.at[iidx.at[pl.ds(_B_OFF, IPW)]],
                              iacc, semi).wait()

    @pl.loop(0, NT)
    def _(k):
        pltpu.make_async_copy(tag_hbm.at[iidx.at[pl.ds(_T_OFF, IPW)]],
                              tacc, semt).wait()

    @pl.loop(0, IPW)
    def _(r):
        for l in range(D // LANE):
            sl = pl.ds(l * LANE, LANE)
            iacc[r, sl] = iacc[r, sl] + tacc[r, sl] * (1.0 / NT)

    pltpu.sync_copy(iacc, i_hbm.at[pl.ds(wid * IPW, IPW)])


_BM = 1024


def _mlp_body(x_ref, w1, b1, w2, b2, w3, b3, w4, b4, o_ref):
    f32 = jnp.float32
    h = jnp.maximum(jnp.dot(x_ref[...], w1[...], preferred_element_type=f32)
                    + b1[...], 0.0)
    h = jnp.maximum(jnp.dot(h, w2[...], preferred_element_type=f32)
                    + b2[...], 0.0)
    h = jnp.maximum(jnp.dot(h, w3[...], preferred_element_type=f32)
                    + b3[...], 0.0)
    o_ref[...] = jnp.dot(h, w4[...], preferred_element_type=f32) + b4[...]


def _user_mlp(x, uW1, ub1, uW2, ub2, uW3, ub3, uW4, ub4):
    full = lambda s: pl.BlockSpec(s, lambda i: (0, 0))
    return pl.pallas_call(
        _mlp_body,
        grid=(B // _BM,),
        in_specs=[
            pl.BlockSpec((_BM, D), lambda i: (i, 0)),
            full(uW1.shape), full(ub1.shape),
            full(uW2.shape), full(ub2.shape),
            full(uW3.shape), full(ub3.shape),
            full(uW4.shape), full(ub4.shape),
        ],
        out_specs=pl.BlockSpec((_BM, D), lambda i: (i, 0)),
        out_shape=jax.ShapeDtypeStruct((B, D), jnp.float32),
        compiler_params=pltpu.CompilerParams(
            dimension_semantics=("arbitrary",)),
    )(x, uW1, ub1, uW2, ub2, uW3, ub3, uW4, ub4)


def _combine_body(u_ref, ip_ref, dn_ref, w1, b1, w2, b2, o_ref):
    f32 = jnp.float32
    h = jnp.maximum(jnp.dot(dn_ref[...], w1[...], preferred_element_type=f32)
                    + b1[...], 0.0)
    d = jnp.dot(h, w2[...], preferred_element_type=f32) + b2[...]
    o_ref[...] = jnp.sum(u_ref[...] * (ip_ref[...] + d), axis=1,
                         keepdims=True)


def _combine(u, ipart, dense8, dW1p, db1, dW2, db2):
    full = lambda s: pl.BlockSpec(s, lambda i: (0, 0))
    return pl.pallas_call(
        _combine_body,
        grid=(B // _BM,),
        in_specs=[
            pl.BlockSpec((_BM, D), lambda i: (i, 0)),
            pl.BlockSpec((_BM, D), lambda i: (i, 0)),
            pl.BlockSpec((_BM, 8), lambda i: (i, 0)),
            full(dW1p.shape), full(db1.shape),
            full(dW2.shape), full(db2.shape),
        ],
        out_specs=pl.BlockSpec((_BM, 1), lambda i: (i, 0)),
        out_shape=jax.ShapeDtypeStruct((B, 1), jnp.float32),
        compiler_params=pltpu.CompilerParams(
            dimension_semantics=("arbitrary",)),
    )(u, ipart, dense8, dW1p, db1, dW2, db2)


def kernel(hist_ids, wish_ids, bid, auth, lang, tags, dense, book_emb,
           auth_emb, lang_emb, tag_emb, dW1, db1, dW2, db2, uW1, ub1,
           uW2, ub2, uW3, ub3, uW4, ub4):
    i32 = jnp.int32
    hist_r = hist_ids.astype(i32).reshape(NWRK, IPW, NH).transpose(0, 2, 1) \
        .reshape(NWRK, NH * IPW)
    wish_r = wish_ids.astype(i32).reshape(NWRK, IPW, NWI).transpose(0, 2, 1) \
        .reshape(NWRK, NWI * IPW)
    tags_t = tags.astype(i32).reshape(NWRK, IPW, NT).transpose(0, 2, 1) \
        .reshape(NWRK, NT * IPW)
    itm = jnp.concatenate(
        [tags_t,
         bid.astype(i32).reshape(NWRK, IPW),
         auth.astype(i32).reshape(NWRK, IPW),
         lang.astype(i32).reshape(NWRK, IPW)],
        axis=1,
    )

    x = _user_pool(hist_r, wish_r, book_emb)
    ipart = _item_pool(itm, book_emb, auth_emb, lang_emb, tag_emb)

    u = _user_mlp(x, uW1, ub1.reshape(1, -1), uW2, ub2.reshape(1, -1),
                  uW3, ub3.reshape(1, -1), uW4, ub4.reshape(1, -1))

    dense8 = jnp.pad(dense, ((0, 0), (0, 5)))
    dW1p = jnp.pad(dW1, ((0, 5), (0, 0)))
    return _combine(u, ipart, dense8, dW1p, db1.reshape(1, -1), dW2,
                    db2.reshape(1, -1))
